# baseline (speedup 1.0000x reference)
import functools

import jax
import jax.numpy as jnp
from jax import lax
from jax.experimental import pallas as pl
from jax.experimental.pallas import tpu as pltpu
from jax.experimental.pallas import tpu_sc as plsc

_K = 1024
_D = 64
_BETA = 0.25
_BT = 512


def _vq_main(x_ref, w_ref, loss_ref, perp_ref, enc_ref, idx_ref, wpad_ref,
             hist_acc, loss_acc, w2_acc, *, n_tokens, n_blocks):
    i = pl.program_id(0)
    xb = x_ref[...]
    w = w_ref[...]

    @pl.when(i == 0)
    def _w2_once():
        w2_acc[...] = jnp.sum(w * w, axis=1, keepdims=True).T
        wpad_ref[...] = jnp.concatenate(
            [w, jnp.zeros((_K, 128 - _D), jnp.float32)], axis=1)

    x2 = jnp.sum(xb * xb, axis=1, keepdims=True)
    xm2 = xb * (-2.0)
    xw2 = jax.lax.dot_general(xm2, w, (((1,), (1,)), ((), ())),
                              preferred_element_type=jnp.float32)
    dist = (x2 + xw2) + w2_acc[...]

    dmin = jnp.min(dist, axis=1, keepdims=True)
    cols = jax.lax.broadcasted_iota(jnp.int32, (_BT, _K), 1)
    idx = jnp.min(jnp.where(dist == dmin, cols, _K),
                  axis=1, keepdims=True)

    enc = (cols == idx).astype(jnp.float32)
    enc_ref[...] = enc
    idx_ref[...] = idx

    block_hist = jax.lax.dot_general(
        jnp.ones((1, _BT), jnp.float32), enc, (((1,), (0,)), ((), ())),
        preferred_element_type=jnp.float32)
    block_loss = jnp.sum(dmin).reshape(1, 1)

    @pl.when(i == 0)
    def _init():
        hist_acc[...] = block_hist
        loss_acc[...] = block_loss

    @pl.when(i != 0)
    def _accum():
        hist_acc[...] += block_hist
        loss_acc[...] += block_loss

    @pl.when(i == n_blocks - 1)
    def _finalize():
        loss_ref[...] = loss_acc[...] * ((1.0 + _BETA) / (n_tokens * _D))
        avg = hist_acc[...] / n_tokens
        perp_ref[...] = jnp.exp(-jnp.sum(avg * jnp.log(avg + 1e-10))).reshape(1, 1)


def _tc_stage(flat, W):
    n_tokens = flat.shape[0]
    n_blocks = n_tokens // _BT
    kfn = functools.partial(_vq_main, n_tokens=n_tokens, n_blocks=n_blocks)
    return pl.pallas_call(
        kfn,
        grid=(n_blocks,),
        in_specs=[
            pl.BlockSpec((_BT, _D), lambda i: (i, 0)),
            pl.BlockSpec((_K, _D), lambda i: (0, 0)),
        ],
        out_specs=[
            pl.BlockSpec((1, 1), lambda i: (0, 0)),
            pl.BlockSpec((1, 1), lambda i: (0, 0)),
            pl.BlockSpec((_BT, _K), lambda i: (i, 0)),
            pl.BlockSpec((_BT, 1), lambda i: (i, 0)),
            pl.BlockSpec((_K, 128), lambda i: (0, 0)),
        ],
        out_shape=[
            jax.ShapeDtypeStruct((1, 1), jnp.float32),
            jax.ShapeDtypeStruct((1, 1), jnp.float32),
            jax.ShapeDtypeStruct((n_tokens, _K), jnp.float32),
            jax.ShapeDtypeStruct((n_tokens, 1), jnp.int32),
            jax.ShapeDtypeStruct((_K, 128), jnp.float32),
        ],
        scratch_shapes=[
            pltpu.VMEM((1, _K), jnp.float32),
            pltpu.VMEM((1, 1), jnp.float32),
            pltpu.VMEM((1, _K), jnp.float32),
        ],
        compiler_params=pltpu.CompilerParams(
            dimension_semantics=("arbitrary",),
        ),
    )(flat, W)


def _sc_gather(W_pad, idx):
    info = plsc.get_sparse_core_info()
    nw = info.num_cores * info.num_subcores
    n_tokens = idx.shape[0]
    b_per_w = n_tokens // nw
    chunk = 256
    n_chunks = b_per_w // chunk
    dp = W_pad.shape[1]
    mesh = plsc.VectorSubcoreMesh(core_axis_name="c", subcore_axis_name="s")

    @functools.partial(
        pl.kernel, mesh=mesh,
        out_type=jax.ShapeDtypeStruct((n_tokens, dp), jnp.float32),
        scratch_types=[
            pltpu.VMEM((b_per_w,), jnp.int32),
            pltpu.VMEM((chunk, dp), jnp.float32),
            pltpu.VMEM((chunk, dp), jnp.float32),
            pltpu.SemaphoreType.DMA,
            pltpu.SemaphoreType.DMA,
        ],
    )
    def gather_k(w_hbm, idx_hbm, out_hbm, idx_v, rows_a, rows_b, sem_a, sem_b):
        wid = lax.axis_index("s") * info.num_cores + lax.axis_index("c")
        base = wid * b_per_w
        pltpu.sync_copy(idx_hbm.at[pl.ds(base, b_per_w)], idx_v)
        rows = (rows_a, rows_b)
        sems = (sem_a, sem_b)

        def start(c):
            return pltpu.async_copy(
                w_hbm.at[idx_v.at[pl.ds(c * chunk, chunk)]],
                rows[c % 2], sems[c % 2])

        copies = [start(0), start(1)]
        for c in range(n_chunks):
            copies[c].wait()
            pltpu.sync_copy(rows[c % 2],
                            out_hbm.at[pl.ds(base + c * chunk, chunk)])
            if c + 2 < n_chunks:
                copies.append(start(c + 2))

    return gather_k(W_pad, idx)


def kernel(x, W):
    flat = x.reshape(-1, _D)
    loss, perp, enc, idx, W_pad = _tc_stage(flat, W)
    quantized = _sc_gather(W_pad, idx.reshape(-1))[:, :_D]
    return (loss[0, 0], quantized, perp[0, 0], enc)


# speedup vs baseline: 1.2082x; 1.2082x over previous
import functools

import jax
import jax.numpy as jnp
from jax.experimental import pallas as pl
from jax.experimental.pallas import tpu as pltpu

_K = 1024
_D = 64
_BETA = 0.25
_BT = 1024


def _vq_main(x_ref, w_ref, loss_ref, q_ref, perp_ref, enc_ref,
             hist_acc, loss_acc, w2_acc, *, n_tokens, n_blocks):
    i = pl.program_id(0)
    xb = x_ref[0]
    w = w_ref[...]

    @pl.when(i == 0)
    def _w2_once():
        w2_acc[...] = jnp.sum(w * w, axis=1, keepdims=True).T

    x2 = jnp.sum(xb * xb, axis=1, keepdims=True)
    xm2 = xb * (-2.0)
    xw2 = jax.lax.dot_general(xm2, w, (((1,), (1,)), ((), ())),
                              preferred_element_type=jnp.float32)
    dist = (x2 + xw2) + w2_acc[...]

    dmin = jnp.min(dist, axis=1, keepdims=True)
    cols = jax.lax.broadcasted_iota(jnp.int32, (_BT, _K), 1)
    idx = jnp.min(jnp.where(dist == dmin, cols, _K),
                  axis=1, keepdims=True)

    enc = (cols == idx).astype(jnp.float32)
    enc_ref[...] = enc
    q_ref[...] = jax.lax.dot_general(enc, w, (((1,), (0,)), ((), ())),
                                     preferred_element_type=jnp.float32)

    block_hist = jnp.sum(enc, axis=0, keepdims=True)
    block_loss = jnp.sum(dmin).reshape(1, 1)

    @pl.when(i == 0)
    def _init():
        hist_acc[...] = block_hist
        loss_acc[...] = block_loss

    @pl.when(i != 0)
    def _accum():
        hist_acc[...] += block_hist
        loss_acc[...] += block_loss

    @pl.when(i == n_blocks - 1)
    def _finalize():
        loss_ref[...] = loss_acc[...] * ((1.0 + _BETA) / (n_tokens * _D))
        avg = hist_acc[...] / n_tokens
        perp_ref[...] = jnp.exp(-jnp.sum(avg * jnp.log(avg + 1e-10))).reshape(1, 1)


def kernel(x, W):
    n_tokens = x.shape[0] * x.shape[1]
    n_blocks = n_tokens // _BT
    blocks_per_row = x.shape[1] // _BT

    kfn = functools.partial(_vq_main, n_tokens=n_tokens, n_blocks=n_blocks)
    loss, quantized, perp, enc = pl.pallas_call(
        kfn,
        grid=(n_blocks,),
        in_specs=[
            pl.BlockSpec((1, _BT, _D),
                         lambda i, b=blocks_per_row: (i // b, i % b, 0)),
            pl.BlockSpec((_K, _D), lambda i: (0, 0)),
        ],
        out_specs=[
            pl.BlockSpec((1, 1), lambda i: (0, 0)),
            pl.BlockSpec((_BT, _D), lambda i: (i, 0)),
            pl.BlockSpec((1, 1), lambda i: (0, 0)),
            pl.BlockSpec((_BT, _K), lambda i: (i, 0)),
        ],
        out_shape=[
            jax.ShapeDtypeStruct((1, 1), jnp.float32),
            jax.ShapeDtypeStruct((n_tokens, _D), jnp.float32),
            jax.ShapeDtypeStruct((1, 1), jnp.float32),
            jax.ShapeDtypeStruct((n_tokens, _K), jnp.float32),
        ],
        scratch_shapes=[
            pltpu.VMEM((1, _K), jnp.float32),
            pltpu.VMEM((1, 1), jnp.float32),
            pltpu.VMEM((1, _K), jnp.float32),
        ],
        compiler_params=pltpu.CompilerParams(
            dimension_semantics=("arbitrary",),
        ),
    )(x, W)

    return (loss[0, 0], quantized, perp[0, 0], enc)


# speedup vs baseline: 1.3503x; 1.1177x over previous
import functools

import jax
import jax.numpy as jnp
from jax.experimental import pallas as pl
from jax.experimental.pallas import tpu as pltpu

_K = 1024
_D = 64
_BETA = 0.25
_BT = 2048


def _vq_main(x_ref, w_ref, loss_ref, q_ref, perp_ref, enc_ref,
             hist_acc, loss_acc, w2_acc, *, n_tokens, n_blocks):
    i = pl.program_id(0)
    xb = x_ref[...].reshape(_BT, _D)
    w = w_ref[...]

    @pl.when(i == 0)
    def _w2_once():
        w2_acc[...] = jnp.sum(w * w, axis=1, keepdims=True).T

    x2 = jnp.sum(xb * xb, axis=1, keepdims=True)
    xm2 = xb * (-2.0)
    xw2 = jax.lax.dot_general(xm2, w, (((1,), (1,)), ((), ())),
                              preferred_element_type=jnp.float32)
    dist = (x2 + xw2) + w2_acc[...]

    dmin = jnp.min(dist, axis=1, keepdims=True)
    cols = jax.lax.broadcasted_iota(jnp.int32, (_BT, _K), 1)
    idx = jnp.min(jnp.where(dist == dmin, cols, _K),
                  axis=1, keepdims=True)

    enc = (cols == idx).astype(jnp.float32)
    enc_ref[...] = enc
    q_ref[...] = jax.lax.dot_general(enc, w, (((1,), (0,)), ((), ())),
                                     preferred_element_type=jnp.float32)

    block_hist = jax.lax.dot_general(
        jnp.ones((1, _BT), jnp.float32), enc, (((1,), (0,)), ((), ())),
        preferred_element_type=jnp.float32)
    block_loss = jnp.sum(dmin).reshape(1, 1)

    @pl.when(i == 0)
    def _init():
        hist_acc[...] = block_hist
        loss_acc[...] = block_loss

    @pl.when(i != 0)
    def _accum():
        hist_acc[...] += block_hist
        loss_acc[...] += block_loss

    @pl.when(i == n_blocks - 1)
    def _finalize():
        loss_ref[...] = loss_acc[...] * ((1.0 + _BETA) / (n_tokens * _D))
        avg = hist_acc[...] / n_tokens
        perp_ref[...] = jnp.exp(-jnp.sum(avg * jnp.log(avg + 1e-10))).reshape(1, 1)


def kernel(x, W):
    n_tokens = x.shape[0] * x.shape[1]
    n_blocks = n_tokens // _BT
    blocks_per_row = x.shape[1] // _BT

    kfn = functools.partial(_vq_main, n_tokens=n_tokens, n_blocks=n_blocks)
    loss, quantized, perp, enc = pl.pallas_call(
        kfn,
        grid=(n_blocks,),
        in_specs=[
            pl.BlockSpec((_BT // 1024, 1024, _D), lambda i: (i, 0, 0)),
            pl.BlockSpec((_K, _D), lambda i: (0, 0)),
        ],
        out_specs=[
            pl.BlockSpec((1, 1), lambda i: (0, 0)),
            pl.BlockSpec((_BT, _D), lambda i: (i, 0)),
            pl.BlockSpec((1, 1), lambda i: (0, 0)),
            pl.BlockSpec((_BT, _K), lambda i: (i, 0)),
        ],
        out_shape=[
            jax.ShapeDtypeStruct((1, 1), jnp.float32),
            jax.ShapeDtypeStruct((n_tokens, _D), jnp.float32),
            jax.ShapeDtypeStruct((1, 1), jnp.float32),
            jax.ShapeDtypeStruct((n_tokens, _K), jnp.float32),
        ],
        scratch_shapes=[
            pltpu.VMEM((1, _K), jnp.float32),
            pltpu.VMEM((1, 1), jnp.float32),
            pltpu.VMEM((1, _K), jnp.float32),
        ],
        compiler_params=pltpu.CompilerParams(
            dimension_semantics=("arbitrary",),
        ),
    )(x, W)

    return (loss[0, 0], quantized, perp[0, 0], enc)


# speedup vs baseline: 1.3632x; 1.0095x over previous
import functools

import jax
import jax.numpy as jnp
from jax.experimental import pallas as pl
from jax.experimental.pallas import tpu as pltpu

_K = 1024
_D = 64
_BETA = 0.25
_BT = 4096


def _vq_main(x_ref, w_ref, loss_ref, q_ref, perp_ref, enc_ref,
             hist_acc, loss_acc, w2_acc, *, n_tokens, n_blocks):
    i = pl.program_id(0)
    xb = x_ref[...].reshape(_BT, _D)
    w = w_ref[...]

    @pl.when(i == 0)
    def _w2_once():
        w2_acc[...] = jnp.sum(w * w, axis=1, keepdims=True).T

    x2 = jnp.sum(xb * xb, axis=1, keepdims=True)
    xm2 = xb * (-2.0)
    xw2 = jax.lax.dot_general(xm2, w, (((1,), (1,)), ((), ())),
                              preferred_element_type=jnp.float32)
    dist = (x2 + xw2) + w2_acc[...]

    dmin = jnp.min(dist, axis=1, keepdims=True)
    cols = jax.lax.broadcasted_iota(jnp.int32, (_BT, _K), 1)
    idx = jnp.min(jnp.where(dist == dmin, cols, _K),
                  axis=1, keepdims=True)

    enc = (cols == idx).astype(jnp.float32)
    enc_ref[...] = enc
    q_ref[...] = jax.lax.dot_general(enc, w, (((1,), (0,)), ((), ())),
                                     preferred_element_type=jnp.float32)

    block_hist = jax.lax.dot_general(
        jnp.ones((1, _BT), jnp.float32), enc, (((1,), (0,)), ((), ())),
        preferred_element_type=jnp.float32)
    block_loss = jnp.sum(dmin).reshape(1, 1)

    @pl.when(i == 0)
    def _init():
        hist_acc[...] = block_hist
        loss_acc[...] = block_loss

    @pl.when(i != 0)
    def _accum():
        hist_acc[...] += block_hist
        loss_acc[...] += block_loss

    @pl.when(i == n_blocks - 1)
    def _finalize():
        loss_ref[...] = loss_acc[...] * ((1.0 + _BETA) / (n_tokens * _D))
        avg = hist_acc[...] / n_tokens
        perp_ref[...] = jnp.exp(-jnp.sum(avg * jnp.log(avg + 1e-10))).reshape(1, 1)


def kernel(x, W):
    n_tokens = x.shape[0] * x.shape[1]
    n_blocks = n_tokens // _BT
    blocks_per_row = x.shape[1] // _BT

    kfn = functools.partial(_vq_main, n_tokens=n_tokens, n_blocks=n_blocks)
    loss, quantized, perp, enc = pl.pallas_call(
        kfn,
        grid=(n_blocks,),
        in_specs=[
            pl.BlockSpec((_BT // 1024, 1024, _D), lambda i: (i, 0, 0)),
            pl.BlockSpec((_K, _D), lambda i: (0, 0)),
        ],
        out_specs=[
            pl.BlockSpec((1, 1), lambda i: (0, 0)),
            pl.BlockSpec((_BT, _D), lambda i: (i, 0)),
            pl.BlockSpec((1, 1), lambda i: (0, 0)),
            pl.BlockSpec((_BT, _K), lambda i: (i, 0)),
        ],
        out_shape=[
            jax.ShapeDtypeStruct((1, 1), jnp.float32),
            jax.ShapeDtypeStruct((n_tokens, _D), jnp.float32),
            jax.ShapeDtypeStruct((1, 1), jnp.float32),
            jax.ShapeDtypeStruct((n_tokens, _K), jnp.float32),
        ],
        scratch_shapes=[
            pltpu.VMEM((1, _K), jnp.float32),
            pltpu.VMEM((1, 1), jnp.float32),
            pltpu.VMEM((1, _K), jnp.float32),
        ],
        compiler_params=pltpu.CompilerParams(
            dimension_semantics=("arbitrary",),
        ),
    )(x, W)

    return (loss[0, 0], quantized, perp[0, 0], enc)


# speedup vs baseline: 1.3649x; 1.0012x over previous
import functools

import jax
import jax.numpy as jnp
from jax.experimental import pallas as pl
from jax.experimental.pallas import tpu as pltpu

_K = 1024
_D = 64
_BETA = 0.25
_BT = 4096


def _vq_main(x_ref, w_ref, loss_ref, q_ref, perp_ref, enc_ref,
             hist_acc, loss_acc, w2_acc, *, n_tokens, n_blocks):
    i = pl.program_id(0)
    xb = x_ref[...].reshape(_BT, _D)
    w = w_ref[...]

    @pl.when(i == 0)
    def _w2_once():
        w2_acc[...] = jnp.sum(w * w, axis=1, keepdims=True).T

    x2 = jnp.sum(xb * xb, axis=1, keepdims=True)
    xm2 = xb * (-2.0)
    xw2 = jax.lax.dot_general(xm2, w, (((1,), (1,)), ((), ())),
                              preferred_element_type=jnp.float32)
    dist = (x2 + xw2) + w2_acc[...]

    dmin = jnp.min(dist, axis=1, keepdims=True)
    cols = jax.lax.broadcasted_iota(jnp.int32, (_BT, _K), 1)
    idx = jnp.min(jnp.where(dist == dmin, cols, _K),
                  axis=1, keepdims=True)

    enc = (cols == idx).astype(jnp.float32)
    enc_ref[...] = enc
    q_ref[...] = jax.lax.dot_general(enc, w, (((1,), (0,)), ((), ())),
                                     preferred_element_type=jnp.float32)

    block_hist = jax.lax.dot_general(
        jnp.ones((1, _BT), jnp.float32), enc, (((1,), (0,)), ((), ())),
        preferred_element_type=jnp.float32)
    block_loss = jnp.sum(dmin).reshape(1, 1)

    @pl.when(i == 0)
    def _init():
        hist_acc[...] = block_hist
        loss_acc[...] = block_loss

    @pl.when(i != 0)
    def _accum():
        hist_acc[...] += block_hist
        loss_acc[...] += block_loss

    @pl.when(i == n_blocks - 1)
    def _finalize():
        loss_ref[...] = loss_acc[...] * ((1.0 + _BETA) / (n_tokens * _D))
        avg = hist_acc[...] / n_tokens
        perp_ref[...] = jnp.exp(-jnp.sum(avg * jnp.log(avg + 1e-10))).reshape(1, 1)


def kernel(x, W):
    n_tokens = x.shape[0] * x.shape[1]
    n_blocks = n_tokens // _BT
    kfn = functools.partial(_vq_main, n_tokens=n_tokens, n_blocks=n_blocks)
    loss, quantized, perp, enc = pl.pallas_call(
        kfn,
        grid=(n_blocks,),
        in_specs=[
            pl.BlockSpec((_BT // 1024, 1024, _D), lambda i: (i, 0, 0)),
            pl.BlockSpec((_K, _D), lambda i: (0, 0)),
        ],
        out_specs=[
            pl.BlockSpec((1, 1), lambda i: (0, 0)),
            pl.BlockSpec((_BT, _D), lambda i: (i, 0)),
            pl.BlockSpec((1, 1), lambda i: (0, 0)),
            pl.BlockSpec((_BT, _K), lambda i: (i, 0)),
        ],
        out_shape=[
            jax.ShapeDtypeStruct((1, 1), jnp.float32),
            jax.ShapeDtypeStruct((n_tokens, _D), jnp.float32),
            jax.ShapeDtypeStruct((1, 1), jnp.float32),
            jax.ShapeDtypeStruct((n_tokens, _K), jnp.float32),
        ],
        scratch_shapes=[
            pltpu.VMEM((1, _K), jnp.float32),
            pltpu.VMEM((1, 1), jnp.float32),
            pltpu.VMEM((1, _K), jnp.float32),
        ],
        compiler_params=pltpu.CompilerParams(
            dimension_semantics=("arbitrary",),
        ),
    )(x, W)

    return (loss[0, 0], quantized, perp[0, 0], enc)
